# baseline (speedup 1.0000x reference)
import functools

import jax
import jax.numpy as jnp
from jax import lax
from jax.experimental import pallas as pl
from jax.experimental.pallas import tpu as pltpu
from jax.experimental.pallas import tpu_sc as plsc

NW = 32
CH = 128
STEPS_MP = 6
STEPS_S2S = 6
NUM_G = 64


def _proj_body(nf, w, b, o):
    o[...] = jnp.maximum(
        jnp.dot(nf[...], w[...], preferred_element_type=jnp.float32) + b[...], 0.0)


def _edgez_body(ef, w, b, o):
    o[...] = jnp.maximum(
        jnp.dot(ef[...], w[...], preferred_element_type=jnp.float32) + b[...], 0.0)


def _msg_body(xs, z, wq, br, o, *, te, h, eh):
    x = xs[...]
    p2 = jnp.dot(x, wq[...], preferred_element_type=jnp.float32)
    p3 = p2.reshape(te, h, eh)
    m = jnp.sum(p3 * z[...][:, None, :], axis=-1)
    o[...] = m + jnp.dot(x, br[...], preferred_element_type=jnp.float32)


def _gru_body(aggp, hid, wr, wz, wn, ur, uz, un, br, bz, bni, bnh, bc, o):
    agg = aggp[0] + aggp[1]
    conv = jnp.maximum(agg + bc[...], 0.0)
    h = hid[...]
    dot = lambda a, w: jnp.dot(a, w[...], preferred_element_type=jnp.float32)
    r = jax.nn.sigmoid(dot(conv, wr) + dot(h, ur) + br[...])
    zg = jax.nn.sigmoid(dot(conv, wz) + dot(h, uz) + bz[...])
    n = jnp.tanh(dot(conv, wn) + bni[...] + r * (dot(h, un) + bnh[...]))
    o[...] = (1.0 - zg) * n + zg * h


def _s2s_body(feat_ref, n2g_ref, *refs, v, g, h, steps):
    lw = [refs[l * 12:(l + 1) * 12] for l in range(3)]
    wp_ref, bp_ref, out_ref = refs[36], refs[37], refs[38]
    feat = feat_ref[...]
    n2g = n2g_ref[...]
    gid = lax.broadcasted_iota(jnp.int32, (v, g), 1)
    A = (n2g == gid).astype(jnp.float32)
    dot = lambda a, w: jnp.dot(a, w[...], preferred_element_type=jnp.float32)
    hs = [jnp.zeros((g, h), jnp.float32) for _ in range(3)]
    cs = [jnp.zeros((g, h), jnp.float32) for _ in range(3)]
    q_star = jnp.zeros((g, 2 * h), jnp.float32)
    for _ in range(steps):
        layer_in = q_star
        for l in range(3):
            wi, wf, wg, wo, ui, uf, ug, uo, bi, bf, bg, bo = lw[l]
            ig = jax.nn.sigmoid(dot(layer_in, wi) + dot(hs[l], ui) + bi[...])
            fg = jax.nn.sigmoid(dot(layer_in, wf) + dot(hs[l], uf) + bf[...])
            gg = jnp.tanh(dot(layer_in, wg) + dot(hs[l], ug) + bg[...])
            og = jax.nn.sigmoid(dot(layer_in, wo) + dot(hs[l], uo) + bo[...])
            cs[l] = fg * cs[l] + ig * gg
            hs[l] = og * jnp.tanh(cs[l])
            layer_in = hs[l]
        q = hs[2]
        qn = jnp.dot(A, q, preferred_element_type=jnp.float32)
        e = jnp.sum(feat * qn, axis=-1, keepdims=True)
        me = jnp.where(A > 0, e, -1e30)
        mmax = jnp.max(me, axis=0, keepdims=True)
        emaxv = jnp.sum(A * mmax, axis=1, keepdims=True)
        ex = jnp.exp(e - emaxv)
        denom = jnp.sum(A * ex, axis=0, keepdims=True)
        denv = jnp.sum(A * denom, axis=1, keepdims=True)
        alpha = ex / denv
        readout = lax.dot_general(A, feat * alpha, (((0,), (0,)), ((), ())),
                                  preferred_element_type=jnp.float32)
        q_star = jnp.concatenate([q, readout], axis=1)
    out_ref[...] = jnp.maximum(
        jnp.dot(q_star, wp_ref[...], preferred_element_type=jnp.float32)
        + bp_ref[...], 0.0)


_MESH = plsc.VectorSubcoreMesh(core_axis_name="c", subcore_axis_name="s")


@functools.lru_cache(maxsize=None)
def _make_gather(v, h, e_pad):
    nch_w = e_pad // (NW * CH)

    @functools.partial(
        pl.kernel,
        out_type=jax.ShapeDtypeStruct((e_pad, h), jnp.float32),
        mesh=_MESH,
        scratch_types=[pltpu.VMEM((nch_w, CH), jnp.int32),
                       pltpu.VMEM((CH, h), jnp.float32)],
        compiler_params=pltpu.CompilerParams(use_tc_tiling_on_sc=False),
    )
    def _gather(x_hbm, idx_hbm, xs_hbm, idx_v, rows_v):
        c = lax.axis_index("c")
        s = lax.axis_index("s")
        wid = s * 2 + c
        pltpu.sync_copy(idx_hbm.at[wid], idx_v)

        def body(j, carry):
            pltpu.sync_copy(x_hbm.at[idx_v.at[j]], rows_v)
            pltpu.sync_copy(rows_v, xs_hbm.at[pl.ds((wid * nch_w + j) * CH, CH)])
            return carry

        lax.fori_loop(0, nch_w, body, 0)

    return _gather


@functools.lru_cache(maxsize=None)
def _make_scatter(v, h, e_pad):
    nch_w = e_pad // (NW * CH)
    vrows = -(-(v + 1) // (16 * CH)) * (16 * CH)
    zch = vrows // (16 * CH)
    slab = v // 16

    @functools.partial(
        pl.kernel,
        out_type=jax.ShapeDtypeStruct((2, v, h), jnp.float32),
        mesh=_MESH,
        scratch_types=[pltpu.VMEM((nch_w, CH), jnp.int32),
                       pltpu.VMEM((CH, h), jnp.float32),
                       pltpu.VMEM_SHARED((vrows, h), jnp.float32)],
        compiler_params=pltpu.CompilerParams(use_tc_tiling_on_sc=False),
    )
    def _scatter(m_hbm, idx_hbm, zero_hbm, agg_hbm, idx_v, rows_v, agg_sh):
        c = lax.axis_index("c")
        s = lax.axis_index("s")
        wid = s * 2 + c
        for t in range(zch):
            pltpu.sync_copy(zero_hbm, agg_sh.at[pl.ds((s * zch + t) * CH, CH)])
        plsc.subcore_barrier()
        pltpu.sync_copy(idx_hbm.at[wid], idx_v)

        def body(j, carry):
            pltpu.sync_copy(m_hbm.at[pl.ds((wid * nch_w + j) * CH, CH)], rows_v)
            pltpu.sync_copy(rows_v, agg_sh.at[idx_v.at[j]], add=True)
            return carry

        lax.fori_loop(0, nch_w, body, 0)
        plsc.subcore_barrier()
        pltpu.sync_copy(agg_sh.at[pl.ds(s * slab, slab)],
                        agg_hbm.at[c, pl.ds(s * slab, slab)])

    return _scatter


def _pick_block(n, cands):
    for c in cands:
        if n % c == 0:
            return c
    return n


def kernel(node_feats, edge_feats, edge_index, node2graph, W_proj, b_proj,
           We1, be1, We2, be2, b_conv, gru_Wih, gru_Whh, gru_bih, gru_bhh,
           lstm_Wih0, lstm_Whh0, lstm_bih0, lstm_bhh0,
           lstm_Wih1, lstm_Whh1, lstm_bih1, lstm_bhh1,
           lstm_Wih2, lstm_Whh2, lstm_bih2, lstm_bhh2, Wp, bp):
    f32 = jnp.float32
    V, NODE_IN = node_feats.shape
    E, EDGE_IN = edge_feats.shape
    H = W_proj.shape[1]
    EH = We1.shape[1]
    G = NUM_G

    e_pad = -(-E // (NW * CH)) * (NW * CH)
    nch_w = e_pad // (NW * CH)
    src = edge_index[0]
    dst = edge_index[1]
    src_p = jnp.concatenate([src, jnp.zeros((e_pad - E,), jnp.int32)])
    dst_p = jnp.concatenate([dst, jnp.full((e_pad - E,), V, jnp.int32)])
    src_w = src_p.reshape(NW, nch_w, CH)
    dst_w = dst_p.reshape(NW, nch_w, CH)
    ef_p = jnp.concatenate(
        [edge_feats, jnp.zeros((e_pad - E, EDGE_IN), f32)]).astype(f32)
    zero_rows = jnp.zeros((CH, H), f32)

    We2q = We2.reshape(EH, H, H).transpose(1, 2, 0).reshape(H, H * EH)
    Br = be2.reshape(H, H)
    b_proj2 = b_proj.reshape(1, H)
    be1_2 = be1.reshape(1, EH)
    bc2 = b_conv.reshape(1, H)
    Wr, Wz, Wn = (gru_Wih[:H].T, gru_Wih[H:2 * H].T, gru_Wih[2 * H:].T)
    Ur, Uz, Un = (gru_Whh[:H].T, gru_Whh[H:2 * H].T, gru_Whh[2 * H:].T)
    br2 = (gru_bih[:H] + gru_bhh[:H]).reshape(1, H)
    bz2 = (gru_bih[H:2 * H] + gru_bhh[H:2 * H]).reshape(1, H)
    bni2 = gru_bih[2 * H:].reshape(1, H)
    bnh2 = gru_bhh[2 * H:].reshape(1, H)

    bvp = _pick_block(V, (1000, 500, 250, 200, 125, 100, 50, 25, 10, 8))
    h0 = pl.pallas_call(
        _proj_body,
        grid=(V // bvp,),
        in_specs=[pl.BlockSpec((bvp, NODE_IN), lambda i: (i, 0)),
                  pl.BlockSpec((NODE_IN, H), lambda i: (0, 0)),
                  pl.BlockSpec((1, H), lambda i: (0, 0))],
        out_specs=pl.BlockSpec((bvp, H), lambda i: (i, 0)),
        out_shape=jax.ShapeDtypeStruct((V, H), f32),
    )(node_feats.astype(f32), W_proj, b_proj2)

    bez = 2048
    z = pl.pallas_call(
        _edgez_body,
        grid=(e_pad // bez,),
        in_specs=[pl.BlockSpec((bez, EDGE_IN), lambda i: (i, 0)),
                  pl.BlockSpec((EDGE_IN, EH), lambda i: (0, 0)),
                  pl.BlockSpec((1, EH), lambda i: (0, 0))],
        out_specs=pl.BlockSpec((bez, EH), lambda i: (i, 0)),
        out_shape=jax.ShapeDtypeStruct((e_pad, EH), f32),
    )(ef_p, We1, be1_2)

    gather = _make_gather(V, H, e_pad)
    scatter = _make_scatter(V, H, e_pad)

    te = 128
    msg_call = pl.pallas_call(
        functools.partial(_msg_body, te=te, h=H, eh=EH),
        grid=(e_pad // te,),
        in_specs=[pl.BlockSpec((te, H), lambda i: (i, 0)),
                  pl.BlockSpec((te, EH), lambda i: (i, 0)),
                  pl.BlockSpec((H, H * EH), lambda i: (0, 0)),
                  pl.BlockSpec((H, H), lambda i: (0, 0))],
        out_specs=pl.BlockSpec((te, H), lambda i: (i, 0)),
        out_shape=jax.ShapeDtypeStruct((e_pad, H), f32),
    )

    bvg = _pick_block(V, (1000, 500, 250, 200, 125, 100, 50, 25, 10, 8))
    gru_call = pl.pallas_call(
        _gru_body,
        grid=(V // bvg,),
        in_specs=[pl.BlockSpec((2, bvg, H), lambda i: (0, i, 0)),
                  pl.BlockSpec((bvg, H), lambda i: (i, 0))]
                 + [pl.BlockSpec((H, H), lambda i: (0, 0))] * 6
                 + [pl.BlockSpec((1, H), lambda i: (0, 0))] * 5,
        out_specs=pl.BlockSpec((bvg, H), lambda i: (i, 0)),
        out_shape=jax.ShapeDtypeStruct((V, H), f32),
    )

    hidden = h0
    x = h0
    for _ in range(STEPS_MP):
        xs = gather(x, src_w)
        m = msg_call(xs, z, We2q, Br)
        aggp = scatter(m, dst_w, zero_rows)
        hidden = gru_call(aggp, hidden, Wr, Wz, Wn, Ur, Uz, Un,
                          br2, bz2, bni2, bnh2, bc2)
        x = hidden

    lstm = [(lstm_Wih0, lstm_Whh0, lstm_bih0, lstm_bhh0),
            (lstm_Wih1, lstm_Whh1, lstm_bih1, lstm_bhh1),
            (lstm_Wih2, lstm_Whh2, lstm_bih2, lstm_bhh2)]
    s2s_args = [x, node2graph.reshape(V, 1)]
    for wih, whh, bih, bhh in lstm:
        for j in range(4):
            s2s_args.append(wih[j * H:(j + 1) * H].T)
        for j in range(4):
            s2s_args.append(whh[j * H:(j + 1) * H].T)
        for j in range(4):
            s2s_args.append((bih[j * H:(j + 1) * H]
                             + bhh[j * H:(j + 1) * H]).reshape(1, H))
    s2s_args += [Wp, bp.reshape(1, H)]
    out = pl.pallas_call(
        functools.partial(_s2s_body, v=V, g=G, h=H, steps=STEPS_S2S),
        out_shape=jax.ShapeDtypeStruct((G, H), f32),
    )(*s2s_args)
    return out


# speedup vs baseline: 1.0538x; 1.0538x over previous
import functools

import jax
import jax.numpy as jnp
from jax import lax
from jax.experimental import pallas as pl
from jax.experimental.pallas import tpu as pltpu
from jax.experimental.pallas import tpu_sc as plsc

NW = 32
CH = 128
STEPS_MP = 6
STEPS_S2S = 6
NUM_G = 64


def _proj_body(nf, w, b, o):
    o[...] = jnp.maximum(
        jnp.dot(nf[...], w[...], preferred_element_type=jnp.float32) + b[...], 0.0)


def _edgez_body(ef, w, b, o):
    o[...] = jnp.maximum(
        jnp.dot(ef[...], w[...], preferred_element_type=jnp.float32) + b[...], 0.0)


def _msg_body(xs, z, wq, br, o, *, te, h, eh):
    x = xs[...]
    xb = x.astype(jnp.bfloat16)
    p2 = jnp.dot(xb, wq[...], preferred_element_type=jnp.float32)
    p3 = p2.reshape(te, h, eh)
    m = jnp.sum(p3 * z[...][:, None, :], axis=-1)
    o[...] = m + jnp.dot(x, br[...], preferred_element_type=jnp.float32)


def _gru_body(aggp, hid, wr, wz, wn, ur, uz, un, br, bz, bni, bnh, bc, o):
    agg = aggp[0] + aggp[1]
    conv = jnp.maximum(agg + bc[...], 0.0)
    h = hid[...]
    dot = lambda a, w: jnp.dot(a, w[...], preferred_element_type=jnp.float32)
    r = jax.nn.sigmoid(dot(conv, wr) + dot(h, ur) + br[...])
    zg = jax.nn.sigmoid(dot(conv, wz) + dot(h, uz) + bz[...])
    n = jnp.tanh(dot(conv, wn) + bni[...] + r * (dot(h, un) + bnh[...]))
    o[...] = (1.0 - zg) * n + zg * h


def _s2s_body(feat_ref, n2g_ref, *refs, v, g, h, steps):
    lw = [refs[l * 12:(l + 1) * 12] for l in range(3)]
    wp_ref, bp_ref, out_ref = refs[36], refs[37], refs[38]
    feat = feat_ref[...]
    n2g = n2g_ref[...]
    gid = lax.broadcasted_iota(jnp.int32, (v, g), 1)
    A = (n2g == gid).astype(jnp.float32)
    dot = lambda a, w: jnp.dot(a, w[...], preferred_element_type=jnp.float32)
    hs = [jnp.zeros((g, h), jnp.float32) for _ in range(3)]
    cs = [jnp.zeros((g, h), jnp.float32) for _ in range(3)]
    q_star = jnp.zeros((g, 2 * h), jnp.float32)
    for _ in range(steps):
        layer_in = q_star
        for l in range(3):
            wi, wf, wg, wo, ui, uf, ug, uo, bi, bf, bg, bo = lw[l]
            ig = jax.nn.sigmoid(dot(layer_in, wi) + dot(hs[l], ui) + bi[...])
            fg = jax.nn.sigmoid(dot(layer_in, wf) + dot(hs[l], uf) + bf[...])
            gg = jnp.tanh(dot(layer_in, wg) + dot(hs[l], ug) + bg[...])
            og = jax.nn.sigmoid(dot(layer_in, wo) + dot(hs[l], uo) + bo[...])
            cs[l] = fg * cs[l] + ig * gg
            hs[l] = og * jnp.tanh(cs[l])
            layer_in = hs[l]
        q = hs[2]
        qn = jnp.dot(A, q, preferred_element_type=jnp.float32)
        e = jnp.sum(feat * qn, axis=-1, keepdims=True)
        me = jnp.where(A > 0, e, -1e30)
        mmax = jnp.max(me, axis=0, keepdims=True)
        emaxv = jnp.sum(A * mmax, axis=1, keepdims=True)
        ex = jnp.exp(e - emaxv)
        denom = jnp.sum(A * ex, axis=0, keepdims=True)
        denv = jnp.sum(A * denom, axis=1, keepdims=True)
        alpha = ex / denv
        readout = lax.dot_general(A, feat * alpha, (((0,), (0,)), ((), ())),
                                  preferred_element_type=jnp.float32)
        q_star = jnp.concatenate([q, readout], axis=1)
    out_ref[...] = jnp.maximum(
        jnp.dot(q_star, wp_ref[...], preferred_element_type=jnp.float32)
        + bp_ref[...], 0.0)


_MESH = plsc.VectorSubcoreMesh(core_axis_name="c", subcore_axis_name="s")


@functools.lru_cache(maxsize=None)
def _make_gather(v, h, e_pad):
    nch_w = e_pad // (NW * CH)

    @functools.partial(
        pl.kernel,
        out_type=jax.ShapeDtypeStruct((e_pad, h), jnp.float32),
        mesh=_MESH,
        scratch_types=[pltpu.VMEM((nch_w, CH), jnp.int32),
                       pltpu.VMEM((CH, h), jnp.float32)],
        compiler_params=pltpu.CompilerParams(use_tc_tiling_on_sc=False),
    )
    def _gather(x_hbm, idx_hbm, xs_hbm, idx_v, rows_v):
        c = lax.axis_index("c")
        s = lax.axis_index("s")
        wid = s * 2 + c
        pltpu.sync_copy(idx_hbm.at[wid], idx_v)

        def body(j, carry):
            pltpu.sync_copy(x_hbm.at[idx_v.at[j]], rows_v)
            pltpu.sync_copy(rows_v, xs_hbm.at[pl.ds((wid * nch_w + j) * CH, CH)])
            return carry

        lax.fori_loop(0, nch_w, body, 0)

    return _gather


@functools.lru_cache(maxsize=None)
def _make_scatter(v, h, e_pad):
    nch_w = e_pad // (NW * CH)
    vrows = -(-(v + 1) // (16 * CH)) * (16 * CH)
    zch = vrows // (16 * CH)
    slab = v // 16

    @functools.partial(
        pl.kernel,
        out_type=jax.ShapeDtypeStruct((2, v, h), jnp.float32),
        mesh=_MESH,
        scratch_types=[pltpu.VMEM((nch_w, CH), jnp.int32),
                       pltpu.VMEM((CH, h), jnp.float32),
                       pltpu.VMEM_SHARED((vrows, h), jnp.float32)],
        compiler_params=pltpu.CompilerParams(use_tc_tiling_on_sc=False),
    )
    def _scatter(m_hbm, idx_hbm, zero_hbm, agg_hbm, idx_v, rows_v, agg_sh):
        c = lax.axis_index("c")
        s = lax.axis_index("s")
        wid = s * 2 + c
        for t in range(zch):
            pltpu.sync_copy(zero_hbm, agg_sh.at[pl.ds((s * zch + t) * CH, CH)])
        plsc.subcore_barrier()
        pltpu.sync_copy(idx_hbm.at[wid], idx_v)

        def body(j, carry):
            pltpu.sync_copy(m_hbm.at[pl.ds((wid * nch_w + j) * CH, CH)], rows_v)
            pltpu.sync_copy(rows_v, agg_sh.at[idx_v.at[j]], add=True)
            return carry

        lax.fori_loop(0, nch_w, body, 0)
        plsc.subcore_barrier()
        pltpu.sync_copy(agg_sh.at[pl.ds(s * slab, slab)],
                        agg_hbm.at[c, pl.ds(s * slab, slab)])

    return _scatter


def _pick_block(n, cands):
    for c in cands:
        if n % c == 0:
            return c
    return n


def kernel(node_feats, edge_feats, edge_index, node2graph, W_proj, b_proj,
           We1, be1, We2, be2, b_conv, gru_Wih, gru_Whh, gru_bih, gru_bhh,
           lstm_Wih0, lstm_Whh0, lstm_bih0, lstm_bhh0,
           lstm_Wih1, lstm_Whh1, lstm_bih1, lstm_bhh1,
           lstm_Wih2, lstm_Whh2, lstm_bih2, lstm_bhh2, Wp, bp):
    f32 = jnp.float32
    V, NODE_IN = node_feats.shape
    E, EDGE_IN = edge_feats.shape
    H = W_proj.shape[1]
    EH = We1.shape[1]
    G = NUM_G

    e_pad = -(-E // (NW * CH)) * (NW * CH)
    nch_w = e_pad // (NW * CH)
    src = edge_index[0]
    dst = edge_index[1]
    src_p = jnp.concatenate([src, jnp.zeros((e_pad - E,), jnp.int32)])
    dst_p = jnp.concatenate([dst, jnp.full((e_pad - E,), V, jnp.int32)])
    src_w = src_p.reshape(NW, nch_w, CH)
    dst_w = dst_p.reshape(NW, nch_w, CH)
    ef_p = jnp.concatenate(
        [edge_feats, jnp.zeros((e_pad - E, EDGE_IN), f32)]).astype(f32)
    zero_rows = jnp.zeros((CH, H), f32)

    We2q = We2.reshape(EH, H, H).transpose(1, 2, 0).reshape(H, H * EH)
    We2q = We2q.astype(jnp.bfloat16)
    Br = be2.reshape(H, H)
    b_proj2 = b_proj.reshape(1, H)
    be1_2 = be1.reshape(1, EH)
    bc2 = b_conv.reshape(1, H)
    Wr, Wz, Wn = (gru_Wih[:H].T, gru_Wih[H:2 * H].T, gru_Wih[2 * H:].T)
    Ur, Uz, Un = (gru_Whh[:H].T, gru_Whh[H:2 * H].T, gru_Whh[2 * H:].T)
    br2 = (gru_bih[:H] + gru_bhh[:H]).reshape(1, H)
    bz2 = (gru_bih[H:2 * H] + gru_bhh[H:2 * H]).reshape(1, H)
    bni2 = gru_bih[2 * H:].reshape(1, H)
    bnh2 = gru_bhh[2 * H:].reshape(1, H)

    bvp = _pick_block(V, (1000, 500, 250, 200, 125, 100, 50, 25, 10, 8))
    h0 = pl.pallas_call(
        _proj_body,
        grid=(V // bvp,),
        in_specs=[pl.BlockSpec((bvp, NODE_IN), lambda i: (i, 0)),
                  pl.BlockSpec((NODE_IN, H), lambda i: (0, 0)),
                  pl.BlockSpec((1, H), lambda i: (0, 0))],
        out_specs=pl.BlockSpec((bvp, H), lambda i: (i, 0)),
        out_shape=jax.ShapeDtypeStruct((V, H), f32),
    )(node_feats.astype(f32), W_proj, b_proj2)

    bez = 2048
    z = pl.pallas_call(
        _edgez_body,
        grid=(e_pad // bez,),
        in_specs=[pl.BlockSpec((bez, EDGE_IN), lambda i: (i, 0)),
                  pl.BlockSpec((EDGE_IN, EH), lambda i: (0, 0)),
                  pl.BlockSpec((1, EH), lambda i: (0, 0))],
        out_specs=pl.BlockSpec((bez, EH), lambda i: (i, 0)),
        out_shape=jax.ShapeDtypeStruct((e_pad, EH), f32),
    )(ef_p, We1, be1_2)

    gather = _make_gather(V, H, e_pad)
    scatter = _make_scatter(V, H, e_pad)

    te = 512
    msg_call = pl.pallas_call(
        functools.partial(_msg_body, te=te, h=H, eh=EH),
        grid=(e_pad // te,),
        in_specs=[pl.BlockSpec((te, H), lambda i: (i, 0)),
                  pl.BlockSpec((te, EH), lambda i: (i, 0)),
                  pl.BlockSpec((H, H * EH), lambda i: (0, 0)),
                  pl.BlockSpec((H, H), lambda i: (0, 0))],
        out_specs=pl.BlockSpec((te, H), lambda i: (i, 0)),
        out_shape=jax.ShapeDtypeStruct((e_pad, H), f32),
        compiler_params=pltpu.CompilerParams(
            dimension_semantics=("arbitrary",)),
    )

    bvg = _pick_block(V, (1000, 500, 250, 200, 125, 100, 50, 25, 10, 8))
    gru_call = pl.pallas_call(
        _gru_body,
        grid=(V // bvg,),
        in_specs=[pl.BlockSpec((2, bvg, H), lambda i: (0, i, 0)),
                  pl.BlockSpec((bvg, H), lambda i: (i, 0))]
                 + [pl.BlockSpec((H, H), lambda i: (0, 0))] * 6
                 + [pl.BlockSpec((1, H), lambda i: (0, 0))] * 5,
        out_specs=pl.BlockSpec((bvg, H), lambda i: (i, 0)),
        out_shape=jax.ShapeDtypeStruct((V, H), f32),
    )

    hidden = h0
    x = h0
    for _ in range(STEPS_MP):
        xs = gather(x, src_w)
        m = msg_call(xs, z, We2q, Br)
        aggp = scatter(m, dst_w, zero_rows)
        hidden = gru_call(aggp, hidden, Wr, Wz, Wn, Ur, Uz, Un,
                          br2, bz2, bni2, bnh2, bc2)
        x = hidden

    lstm = [(lstm_Wih0, lstm_Whh0, lstm_bih0, lstm_bhh0),
            (lstm_Wih1, lstm_Whh1, lstm_bih1, lstm_bhh1),
            (lstm_Wih2, lstm_Whh2, lstm_bih2, lstm_bhh2)]
    s2s_args = [x, node2graph.reshape(V, 1)]
    for wih, whh, bih, bhh in lstm:
        for j in range(4):
            s2s_args.append(wih[j * H:(j + 1) * H].T)
        for j in range(4):
            s2s_args.append(whh[j * H:(j + 1) * H].T)
        for j in range(4):
            s2s_args.append((bih[j * H:(j + 1) * H]
                             + bhh[j * H:(j + 1) * H]).reshape(1, H))
    s2s_args += [Wp, bp.reshape(1, H)]
    out = pl.pallas_call(
        functools.partial(_s2s_body, v=V, g=G, h=H, steps=STEPS_S2S),
        out_shape=jax.ShapeDtypeStruct((G, H), f32),
    )(*s2s_args)
    return out


# speedup vs baseline: 2.8068x; 2.6635x over previous
import functools

import jax
import jax.numpy as jnp
from jax import lax
from jax.experimental import pallas as pl
from jax.experimental.pallas import tpu as pltpu
from jax.experimental.pallas import tpu_sc as plsc

NW = 32
CH = 128
STEPS_MP = 6
STEPS_S2S = 6
NUM_G = 64


def _proj_body(nf, w, b, o):
    o[...] = jnp.maximum(
        jnp.dot(nf[...], w[...], preferred_element_type=jnp.float32) + b[...], 0.0)


def _edgez_body(ef, w, b, o):
    o[...] = jnp.maximum(
        jnp.dot(ef[...], w[...], preferred_element_type=jnp.float32) + b[...],
        0.0).astype(jnp.bfloat16)


def _msg_body(xs, z, wq, br, ssum, o, *, te, h, eh):
    x = xs[...]
    xb = x.astype(jnp.bfloat16)
    p2 = jnp.dot(xb, wq[...],
                 preferred_element_type=jnp.float32).astype(jnp.bfloat16)
    zt = jnp.concatenate([z[...]] * h, axis=1)
    m = jnp.dot(p2 * zt, ssum[...], preferred_element_type=jnp.float32)
    o[...] = m + jnp.dot(x, br[...], preferred_element_type=jnp.float32)


def _gru_body(aggp, hid, wr, wz, wn, ur, uz, un, br, bz, bni, bnh, bc, o):
    agg = aggp[0] + aggp[1]
    conv = jnp.maximum(agg + bc[...], 0.0)
    h = hid[...]
    dot = lambda a, w: jnp.dot(a, w[...], preferred_element_type=jnp.float32)
    r = jax.nn.sigmoid(dot(conv, wr) + dot(h, ur) + br[...])
    zg = jax.nn.sigmoid(dot(conv, wz) + dot(h, uz) + bz[...])
    n = jnp.tanh(dot(conv, wn) + bni[...] + r * (dot(h, un) + bnh[...]))
    o[...] = (1.0 - zg) * n + zg * h


def _s2s_body(feat_ref, n2g_ref, *refs, v, g, h, steps):
    lw = [refs[l * 12:(l + 1) * 12] for l in range(3)]
    wp_ref, bp_ref, out_ref = refs[36], refs[37], refs[38]
    feat = feat_ref[...]
    n2g = n2g_ref[...]
    gid = lax.broadcasted_iota(jnp.int32, (v, g), 1)
    A = (n2g == gid).astype(jnp.float32)
    dot = lambda a, w: jnp.dot(a, w[...], preferred_element_type=jnp.float32)
    hs = [jnp.zeros((g, h), jnp.float32) for _ in range(3)]
    cs = [jnp.zeros((g, h), jnp.float32) for _ in range(3)]
    q_star = jnp.zeros((g, 2 * h), jnp.float32)
    for _ in range(steps):
        layer_in = q_star
        for l in range(3):
            wi, wf, wg, wo, ui, uf, ug, uo, bi, bf, bg, bo = lw[l]
            ig = jax.nn.sigmoid(dot(layer_in, wi) + dot(hs[l], ui) + bi[...])
            fg = jax.nn.sigmoid(dot(layer_in, wf) + dot(hs[l], uf) + bf[...])
            gg = jnp.tanh(dot(layer_in, wg) + dot(hs[l], ug) + bg[...])
            og = jax.nn.sigmoid(dot(layer_in, wo) + dot(hs[l], uo) + bo[...])
            cs[l] = fg * cs[l] + ig * gg
            hs[l] = og * jnp.tanh(cs[l])
            layer_in = hs[l]
        q = hs[2]
        qn = jnp.dot(A, q, preferred_element_type=jnp.float32)
        e = jnp.sum(feat * qn, axis=-1, keepdims=True)
        me = jnp.where(A > 0, e, -1e30)
        mmax = jnp.max(me, axis=0, keepdims=True)
        emaxv = jnp.sum(A * mmax, axis=1, keepdims=True)
        ex = jnp.exp(e - emaxv)
        denom = jnp.sum(A * ex, axis=0, keepdims=True)
        denv = jnp.sum(A * denom, axis=1, keepdims=True)
        alpha = ex / denv
        readout = lax.dot_general(A, feat * alpha, (((0,), (0,)), ((), ())),
                                  preferred_element_type=jnp.float32)
        q_star = jnp.concatenate([q, readout], axis=1)
    out_ref[...] = jnp.maximum(
        jnp.dot(q_star, wp_ref[...], preferred_element_type=jnp.float32)
        + bp_ref[...], 0.0)


_MESH = plsc.VectorSubcoreMesh(core_axis_name="c", subcore_axis_name="s")


@functools.lru_cache(maxsize=None)
def _make_gather(v, h, e_pad):
    nch_w = e_pad // (NW * CH)

    @functools.partial(
        pl.kernel,
        out_type=jax.ShapeDtypeStruct((e_pad, h), jnp.float32),
        mesh=_MESH,
        scratch_types=[pltpu.VMEM((nch_w, CH), jnp.int32),
                       pltpu.VMEM((CH, h), jnp.float32)],
        compiler_params=pltpu.CompilerParams(use_tc_tiling_on_sc=False),
    )
    def _gather(x_hbm, idx_hbm, xs_hbm, idx_v, rows_v):
        c = lax.axis_index("c")
        s = lax.axis_index("s")
        wid = s * 2 + c
        pltpu.sync_copy(idx_hbm.at[wid], idx_v)

        def body(j, carry):
            pltpu.sync_copy(x_hbm.at[idx_v.at[j]], rows_v)
            pltpu.sync_copy(rows_v, xs_hbm.at[pl.ds((wid * nch_w + j) * CH, CH)])
            return carry

        lax.fori_loop(0, nch_w, body, 0)

    return _gather


@functools.lru_cache(maxsize=None)
def _make_scatter(v, h, e_pad):
    nch_w = e_pad // (NW * CH)
    vrows = -(-(v + 1) // (16 * CH)) * (16 * CH)
    zch = vrows // (16 * CH)
    slab = v // 16

    @functools.partial(
        pl.kernel,
        out_type=jax.ShapeDtypeStruct((2, v, h), jnp.float32),
        mesh=_MESH,
        scratch_types=[pltpu.VMEM((nch_w, CH), jnp.int32),
                       pltpu.VMEM((CH, h), jnp.float32),
                       pltpu.VMEM_SHARED((vrows, h), jnp.float32)],
        compiler_params=pltpu.CompilerParams(use_tc_tiling_on_sc=False),
    )
    def _scatter(m_hbm, idx_hbm, zero_hbm, agg_hbm, idx_v, rows_v, agg_sh):
        c = lax.axis_index("c")
        s = lax.axis_index("s")
        wid = s * 2 + c
        for t in range(zch):
            pltpu.sync_copy(zero_hbm, agg_sh.at[pl.ds((s * zch + t) * CH, CH)])
        plsc.subcore_barrier()
        pltpu.sync_copy(idx_hbm.at[wid], idx_v)

        def body(j, carry):
            pltpu.sync_copy(m_hbm.at[pl.ds((wid * nch_w + j) * CH, CH)], rows_v)
            pltpu.sync_copy(rows_v, agg_sh.at[idx_v.at[j]], add=True)
            return carry

        lax.fori_loop(0, nch_w, body, 0)
        plsc.subcore_barrier()
        pltpu.sync_copy(agg_sh.at[pl.ds(s * slab, slab)],
                        agg_hbm.at[c, pl.ds(s * slab, slab)])

    return _scatter


def _pick_block(n, cands):
    for c in cands:
        if n % c == 0:
            return c
    return n


def kernel(node_feats, edge_feats, edge_index, node2graph, W_proj, b_proj,
           We1, be1, We2, be2, b_conv, gru_Wih, gru_Whh, gru_bih, gru_bhh,
           lstm_Wih0, lstm_Whh0, lstm_bih0, lstm_bhh0,
           lstm_Wih1, lstm_Whh1, lstm_bih1, lstm_bhh1,
           lstm_Wih2, lstm_Whh2, lstm_bih2, lstm_bhh2, Wp, bp):
    f32 = jnp.float32
    V, NODE_IN = node_feats.shape
    E, EDGE_IN = edge_feats.shape
    H = W_proj.shape[1]
    EH = We1.shape[1]
    G = NUM_G

    e_pad = -(-E // (NW * CH)) * (NW * CH)
    nch_w = e_pad // (NW * CH)
    src = edge_index[0]
    dst = edge_index[1]
    src_p = jnp.concatenate([src, jnp.zeros((e_pad - E,), jnp.int32)])
    dst_p = jnp.concatenate([dst, jnp.full((e_pad - E,), V, jnp.int32)])
    src_w = src_p.reshape(NW, nch_w, CH)
    dst_w = dst_p.reshape(NW, nch_w, CH)
    ef_p = jnp.concatenate(
        [edge_feats, jnp.zeros((e_pad - E, EDGE_IN), f32)]).astype(f32)
    zero_rows = jnp.zeros((CH, H), f32)

    We2q = We2.reshape(EH, H, H).transpose(1, 2, 0).reshape(H, H * EH)
    We2q = We2q.astype(jnp.bfloat16)
    Br = be2.reshape(H, H)
    b_proj2 = b_proj.reshape(1, H)
    be1_2 = be1.reshape(1, EH)
    bc2 = b_conv.reshape(1, H)
    Wr, Wz, Wn = (gru_Wih[:H].T, gru_Wih[H:2 * H].T, gru_Wih[2 * H:].T)
    Ur, Uz, Un = (gru_Whh[:H].T, gru_Whh[H:2 * H].T, gru_Whh[2 * H:].T)
    br2 = (gru_bih[:H] + gru_bhh[:H]).reshape(1, H)
    bz2 = (gru_bih[H:2 * H] + gru_bhh[H:2 * H]).reshape(1, H)
    bni2 = gru_bih[2 * H:].reshape(1, H)
    bnh2 = gru_bhh[2 * H:].reshape(1, H)

    bvp = _pick_block(V, (1000, 500, 250, 200, 125, 100, 50, 25, 10, 8))
    h0 = pl.pallas_call(
        _proj_body,
        grid=(V // bvp,),
        in_specs=[pl.BlockSpec((bvp, NODE_IN), lambda i: (i, 0)),
                  pl.BlockSpec((NODE_IN, H), lambda i: (0, 0)),
                  pl.BlockSpec((1, H), lambda i: (0, 0))],
        out_specs=pl.BlockSpec((bvp, H), lambda i: (i, 0)),
        out_shape=jax.ShapeDtypeStruct((V, H), f32),
    )(node_feats.astype(f32), W_proj, b_proj2)

    bez = 2048
    z = pl.pallas_call(
        _edgez_body,
        grid=(e_pad // bez,),
        in_specs=[pl.BlockSpec((bez, EDGE_IN), lambda i: (i, 0)),
                  pl.BlockSpec((EDGE_IN, EH), lambda i: (0, 0)),
                  pl.BlockSpec((1, EH), lambda i: (0, 0))],
        out_specs=pl.BlockSpec((bez, EH), lambda i: (i, 0)),
        out_shape=jax.ShapeDtypeStruct((e_pad, EH), jnp.bfloat16),
    )(ef_p, We1, be1_2)
    ssum = jnp.kron(jnp.eye(H, dtype=f32),
                    jnp.ones((EH, 1), f32)).astype(jnp.bfloat16)

    gather = _make_gather(V, H, e_pad)
    scatter = _make_scatter(V, H, e_pad)

    te = 512
    msg_call = pl.pallas_call(
        functools.partial(_msg_body, te=te, h=H, eh=EH),
        grid=(e_pad // te,),
        in_specs=[pl.BlockSpec((te, H), lambda i: (i, 0)),
                  pl.BlockSpec((te, EH), lambda i: (i, 0)),
                  pl.BlockSpec((H, H * EH), lambda i: (0, 0)),
                  pl.BlockSpec((H, H), lambda i: (0, 0)),
                  pl.BlockSpec((H * EH, H), lambda i: (0, 0))],
        out_specs=pl.BlockSpec((te, H), lambda i: (i, 0)),
        out_shape=jax.ShapeDtypeStruct((e_pad, H), f32),
        compiler_params=pltpu.CompilerParams(
            dimension_semantics=("arbitrary",)),
    )

    bvg = _pick_block(V, (1000, 500, 250, 200, 125, 100, 50, 25, 10, 8))
    gru_call = pl.pallas_call(
        _gru_body,
        grid=(V // bvg,),
        in_specs=[pl.BlockSpec((2, bvg, H), lambda i: (0, i, 0)),
                  pl.BlockSpec((bvg, H), lambda i: (i, 0))]
                 + [pl.BlockSpec((H, H), lambda i: (0, 0))] * 6
                 + [pl.BlockSpec((1, H), lambda i: (0, 0))] * 5,
        out_specs=pl.BlockSpec((bvg, H), lambda i: (i, 0)),
        out_shape=jax.ShapeDtypeStruct((V, H), f32),
    )

    hidden = h0
    x = h0
    for _ in range(STEPS_MP):
        xs = gather(x, src_w)
        m = msg_call(xs, z, We2q, Br, ssum)
        aggp = scatter(m, dst_w, zero_rows)
        hidden = gru_call(aggp, hidden, Wr, Wz, Wn, Ur, Uz, Un,
                          br2, bz2, bni2, bnh2, bc2)
        x = hidden

    lstm = [(lstm_Wih0, lstm_Whh0, lstm_bih0, lstm_bhh0),
            (lstm_Wih1, lstm_Whh1, lstm_bih1, lstm_bhh1),
            (lstm_Wih2, lstm_Whh2, lstm_bih2, lstm_bhh2)]
    s2s_args = [x, node2graph.reshape(V, 1)]
    for wih, whh, bih, bhh in lstm:
        for j in range(4):
            s2s_args.append(wih[j * H:(j + 1) * H].T)
        for j in range(4):
            s2s_args.append(whh[j * H:(j + 1) * H].T)
        for j in range(4):
            s2s_args.append((bih[j * H:(j + 1) * H]
                             + bhh[j * H:(j + 1) * H]).reshape(1, H))
    s2s_args += [Wp, bp.reshape(1, H)]
    out = pl.pallas_call(
        functools.partial(_s2s_body, v=V, g=G, h=H, steps=STEPS_S2S),
        out_shape=jax.ShapeDtypeStruct((G, H), f32),
    )(*s2s_args)
    return out


# speedup vs baseline: 2.8835x; 1.0273x over previous
import functools

import jax
import jax.numpy as jnp
from jax import lax
from jax.experimental import pallas as pl
from jax.experimental.pallas import tpu as pltpu
from jax.experimental.pallas import tpu_sc as plsc

NW = 32
CH = 128
STEPS_MP = 6
STEPS_S2S = 6
NUM_G = 64


def _proj_body(nf, w, b, o):
    o[...] = jnp.maximum(
        jnp.dot(nf[...], w[...], preferred_element_type=jnp.float32) + b[...], 0.0)


def _edgez_body(ef, w, b, o):
    o[...] = jnp.maximum(
        jnp.dot(ef[...], w[...], preferred_element_type=jnp.float32) + b[...],
        0.0).astype(jnp.bfloat16)


def _msg_body(xs, z, wq, br, ssum, o, *, te, h, eh):
    x = xs[...]
    xb = x.astype(jnp.bfloat16)
    p2 = jnp.dot(xb, wq[...],
                 preferred_element_type=jnp.float32).astype(jnp.bfloat16)
    zt = jnp.concatenate([z[...]] * h, axis=1)
    m = jnp.dot(p2 * zt, ssum[...], preferred_element_type=jnp.float32)
    o[...] = m + jnp.dot(x, br[...], preferred_element_type=jnp.float32)


def _gru_body(aggp, hid, wr, wz, wn, ur, uz, un, br, bz, bni, bnh, bc, o):
    agg = aggp[0] + aggp[1]
    conv = jnp.maximum(agg + bc[...], 0.0)
    h = hid[...]
    dot = lambda a, w: jnp.dot(a, w[...], preferred_element_type=jnp.float32)
    r = jax.nn.sigmoid(dot(conv, wr) + dot(h, ur) + br[...])
    zg = jax.nn.sigmoid(dot(conv, wz) + dot(h, uz) + bz[...])
    n = jnp.tanh(dot(conv, wn) + bni[...] + r * (dot(h, un) + bnh[...]))
    o[...] = (1.0 - zg) * n + zg * h


def _s2s_body(feat_ref, n2g_ref, *refs, v, g, h, steps):
    lw = [refs[l * 12:(l + 1) * 12] for l in range(3)]
    wp_ref, bp_ref, out_ref = refs[36], refs[37], refs[38]
    feat = feat_ref[...]
    n2g = n2g_ref[...]
    gid = lax.broadcasted_iota(jnp.int32, (v, g), 1)
    A = (n2g == gid).astype(jnp.float32)
    dot = lambda a, w: jnp.dot(a, w[...], preferred_element_type=jnp.float32)
    hs = [jnp.zeros((g, h), jnp.float32) for _ in range(3)]
    cs = [jnp.zeros((g, h), jnp.float32) for _ in range(3)]
    q_star = jnp.zeros((g, 2 * h), jnp.float32)
    for _ in range(steps):
        layer_in = q_star
        for l in range(3):
            wi, wf, wg, wo, ui, uf, ug, uo, bi, bf, bg, bo = lw[l]
            ig = jax.nn.sigmoid(dot(layer_in, wi) + dot(hs[l], ui) + bi[...])
            fg = jax.nn.sigmoid(dot(layer_in, wf) + dot(hs[l], uf) + bf[...])
            gg = jnp.tanh(dot(layer_in, wg) + dot(hs[l], ug) + bg[...])
            og = jax.nn.sigmoid(dot(layer_in, wo) + dot(hs[l], uo) + bo[...])
            cs[l] = fg * cs[l] + ig * gg
            hs[l] = og * jnp.tanh(cs[l])
            layer_in = hs[l]
        q = hs[2]
        qn = jnp.dot(A, q, preferred_element_type=jnp.float32)
        e = jnp.sum(feat * qn, axis=-1, keepdims=True)
        me = jnp.where(A > 0, e, -1e30)
        mmax = jnp.max(me, axis=0, keepdims=True)
        emaxv = jnp.sum(A * mmax, axis=1, keepdims=True)
        ex = jnp.exp(e - emaxv)
        denom = jnp.sum(A * ex, axis=0, keepdims=True)
        denv = jnp.sum(A * denom, axis=1, keepdims=True)
        alpha = ex / denv
        readout = lax.dot_general(A, feat * alpha, (((0,), (0,)), ((), ())),
                                  preferred_element_type=jnp.float32)
        q_star = jnp.concatenate([q, readout], axis=1)
    out_ref[...] = jnp.maximum(
        jnp.dot(q_star, wp_ref[...], preferred_element_type=jnp.float32)
        + bp_ref[...], 0.0)


_MESH = plsc.VectorSubcoreMesh(core_axis_name="c", subcore_axis_name="s")


@functools.lru_cache(maxsize=None)
def _make_gather(v, h, e_pad):
    nch_w = e_pad // (NW * CH)

    @functools.partial(
        pl.kernel,
        out_type=jax.ShapeDtypeStruct((e_pad, h), jnp.float32),
        mesh=_MESH,
        scratch_types=[pltpu.VMEM((nch_w, CH), jnp.int32),
                       pltpu.VMEM((CH, h), jnp.float32)],
        compiler_params=pltpu.CompilerParams(use_tc_tiling_on_sc=False),
    )
    def _gather(x_hbm, idx_hbm, xs_hbm, idx_v, rows_v):
        c = lax.axis_index("c")
        s = lax.axis_index("s")
        wid = s * 2 + c
        pltpu.sync_copy(idx_hbm.at[wid], idx_v)

        def body(j, carry):
            pltpu.sync_copy(x_hbm.at[idx_v.at[j]], rows_v)
            pltpu.sync_copy(rows_v, xs_hbm.at[pl.ds((wid * nch_w + j) * CH, CH)])
            return carry

        lax.fori_loop(0, nch_w, body, 0)

    return _gather


@functools.lru_cache(maxsize=None)
def _make_scatter(v, h, e_pad):
    nch_w = e_pad // (NW * CH)
    vrows = -(-(v + 1) // (16 * CH)) * (16 * CH)
    zch = vrows // (16 * CH)
    slab = v // 16

    @functools.partial(
        pl.kernel,
        out_type=jax.ShapeDtypeStruct((2, v, h), jnp.float32),
        mesh=_MESH,
        scratch_types=[pltpu.VMEM((nch_w, CH), jnp.int32),
                       pltpu.VMEM((CH, h), jnp.float32),
                       pltpu.VMEM_SHARED((vrows, h), jnp.float32)],
        compiler_params=pltpu.CompilerParams(use_tc_tiling_on_sc=False),
    )
    def _scatter(m_hbm, idx_hbm, zero_hbm, agg_hbm, idx_v, rows_v, agg_sh):
        c = lax.axis_index("c")
        s = lax.axis_index("s")
        wid = s * 2 + c
        for t in range(zch):
            pltpu.sync_copy(zero_hbm, agg_sh.at[pl.ds((s * zch + t) * CH, CH)])
        plsc.subcore_barrier()
        pltpu.sync_copy(idx_hbm.at[wid], idx_v)

        def body(j, carry):
            pltpu.sync_copy(m_hbm.at[pl.ds((wid * nch_w + j) * CH, CH)], rows_v)
            pltpu.sync_copy(rows_v, agg_sh.at[idx_v.at[j]], add=True)
            return carry

        lax.fori_loop(0, nch_w, body, 0)
        plsc.subcore_barrier()
        pltpu.sync_copy(agg_sh.at[pl.ds(s * slab, slab)],
                        agg_hbm.at[c, pl.ds(s * slab, slab)])

    return _scatter


def _pick_block(n, cands):
    for c in cands:
        if n % c == 0:
            return c
    return n


def kernel(node_feats, edge_feats, edge_index, node2graph, W_proj, b_proj,
           We1, be1, We2, be2, b_conv, gru_Wih, gru_Whh, gru_bih, gru_bhh,
           lstm_Wih0, lstm_Whh0, lstm_bih0, lstm_bhh0,
           lstm_Wih1, lstm_Whh1, lstm_bih1, lstm_bhh1,
           lstm_Wih2, lstm_Whh2, lstm_bih2, lstm_bhh2, Wp, bp):
    f32 = jnp.float32
    V, NODE_IN = node_feats.shape
    E, EDGE_IN = edge_feats.shape
    H = W_proj.shape[1]
    EH = We1.shape[1]
    G = NUM_G

    e_pad = -(-E // (NW * CH)) * (NW * CH)
    nch_w = e_pad // (NW * CH)
    src = edge_index[0]
    dst = edge_index[1]
    src_p = jnp.concatenate([src, jnp.zeros((e_pad - E,), jnp.int32)])
    dst_p = jnp.concatenate([dst, jnp.full((e_pad - E,), V, jnp.int32)])
    src_w = src_p.reshape(NW, nch_w, CH)
    dst_w = dst_p.reshape(NW, nch_w, CH)
    ef_p = jnp.concatenate(
        [edge_feats, jnp.zeros((e_pad - E, EDGE_IN), f32)]).astype(f32)
    zero_rows = jnp.zeros((CH, H), f32)

    We2q = We2.reshape(EH, H, H).transpose(1, 2, 0).reshape(H, H * EH)
    We2q = We2q.astype(jnp.bfloat16)
    Br = be2.reshape(H, H)
    b_proj2 = b_proj.reshape(1, H)
    be1_2 = be1.reshape(1, EH)
    bc2 = b_conv.reshape(1, H)
    Wr, Wz, Wn = (gru_Wih[:H].T, gru_Wih[H:2 * H].T, gru_Wih[2 * H:].T)
    Ur, Uz, Un = (gru_Whh[:H].T, gru_Whh[H:2 * H].T, gru_Whh[2 * H:].T)
    br2 = (gru_bih[:H] + gru_bhh[:H]).reshape(1, H)
    bz2 = (gru_bih[H:2 * H] + gru_bhh[H:2 * H]).reshape(1, H)
    bni2 = gru_bih[2 * H:].reshape(1, H)
    bnh2 = gru_bhh[2 * H:].reshape(1, H)

    bvp = _pick_block(V, (1000, 500, 250, 200, 125, 100, 50, 25, 10, 8))
    h0 = pl.pallas_call(
        _proj_body,
        grid=(V // bvp,),
        in_specs=[pl.BlockSpec((bvp, NODE_IN), lambda i: (i, 0)),
                  pl.BlockSpec((NODE_IN, H), lambda i: (0, 0)),
                  pl.BlockSpec((1, H), lambda i: (0, 0))],
        out_specs=pl.BlockSpec((bvp, H), lambda i: (i, 0)),
        out_shape=jax.ShapeDtypeStruct((V, H), f32),
    )(node_feats.astype(f32), W_proj, b_proj2)

    bez = 2048
    z = pl.pallas_call(
        _edgez_body,
        grid=(e_pad // bez,),
        in_specs=[pl.BlockSpec((bez, EDGE_IN), lambda i: (i, 0)),
                  pl.BlockSpec((EDGE_IN, EH), lambda i: (0, 0)),
                  pl.BlockSpec((1, EH), lambda i: (0, 0))],
        out_specs=pl.BlockSpec((bez, EH), lambda i: (i, 0)),
        out_shape=jax.ShapeDtypeStruct((e_pad, EH), jnp.bfloat16),
    )(ef_p, We1, be1_2)
    ssum = jnp.kron(jnp.eye(H, dtype=f32),
                    jnp.ones((EH, 1), f32)).astype(jnp.bfloat16)

    gather = _make_gather(V, H, e_pad)
    scatter = _make_scatter(V, H, e_pad)

    te = 1024
    msg_call = pl.pallas_call(
        functools.partial(_msg_body, te=te, h=H, eh=EH),
        grid=(e_pad // te,),
        in_specs=[pl.BlockSpec((te, H), lambda i: (i, 0)),
                  pl.BlockSpec((te, EH), lambda i: (i, 0)),
                  pl.BlockSpec((H, H * EH), lambda i: (0, 0)),
                  pl.BlockSpec((H, H), lambda i: (0, 0)),
                  pl.BlockSpec((H * EH, H), lambda i: (0, 0))],
        out_specs=pl.BlockSpec((te, H), lambda i: (i, 0)),
        out_shape=jax.ShapeDtypeStruct((e_pad, H), f32),
        compiler_params=pltpu.CompilerParams(
            dimension_semantics=("arbitrary",)),
    )

    bvg = _pick_block(V, (1000, 500, 250, 200, 125, 100, 50, 25, 10, 8))
    gru_call = pl.pallas_call(
        _gru_body,
        grid=(V // bvg,),
        in_specs=[pl.BlockSpec((2, bvg, H), lambda i: (0, i, 0)),
                  pl.BlockSpec((bvg, H), lambda i: (i, 0))]
                 + [pl.BlockSpec((H, H), lambda i: (0, 0))] * 6
                 + [pl.BlockSpec((1, H), lambda i: (0, 0))] * 5,
        out_specs=pl.BlockSpec((bvg, H), lambda i: (i, 0)),
        out_shape=jax.ShapeDtypeStruct((V, H), f32),
    )

    hidden = h0
    x = h0
    for _ in range(STEPS_MP):
        xs = gather(x, src_w)
        m = msg_call(xs, z, We2q, Br, ssum)
        aggp = scatter(m, dst_w, zero_rows)
        hidden = gru_call(aggp, hidden, Wr, Wz, Wn, Ur, Uz, Un,
                          br2, bz2, bni2, bnh2, bc2)
        x = hidden

    lstm = [(lstm_Wih0, lstm_Whh0, lstm_bih0, lstm_bhh0),
            (lstm_Wih1, lstm_Whh1, lstm_bih1, lstm_bhh1),
            (lstm_Wih2, lstm_Whh2, lstm_bih2, lstm_bhh2)]
    s2s_args = [x, node2graph.reshape(V, 1)]
    for wih, whh, bih, bhh in lstm:
        for j in range(4):
            s2s_args.append(wih[j * H:(j + 1) * H].T)
        for j in range(4):
            s2s_args.append(whh[j * H:(j + 1) * H].T)
        for j in range(4):
            s2s_args.append((bih[j * H:(j + 1) * H]
                             + bhh[j * H:(j + 1) * H]).reshape(1, H))
    s2s_args += [Wp, bp.reshape(1, H)]
    out = pl.pallas_call(
        functools.partial(_s2s_body, v=V, g=G, h=H, steps=STEPS_S2S),
        out_shape=jax.ShapeDtypeStruct((G, H), f32),
    )(*s2s_args)
    return out
